# baseline (speedup 1.0000x reference)
import jax
import jax.numpy as jnp
from jax.experimental import pallas as pl
from jax.experimental.pallas import tpu as pltpu

IN_F = 128
OUT_F = 64
ALPHA = 0.2
BI = 256


def _project(x_ref, w_ref, a_ref, whi_ref, vt_ref, pq_ref):
    x = x_ref[...]
    w1 = w_ref[:IN_F, :]
    w2 = w_ref[IN_F:, :]
    a = a_ref[...]
    whi = jnp.dot(x, w1, preferred_element_type=jnp.float32)
    whi_ref[...] = whi
    vt = jax.lax.dot_general(w2, x, (((0,), (1,)), ((), ())),
                             preferred_element_type=jnp.float32)
    vt_ref[...] = vt
    p = jnp.dot(whi, a.T, preferred_element_type=jnp.float32)
    q = jnp.dot(a, vt, preferred_element_type=jnp.float32)
    pq_ref[0:1, :] = ALPHA * q
    pq_ref[1:2, :] = ALPHA * p.T


def _attend(whi_ref, whib_ref, vt_ref, pq_ref, adj_ref, a_ref, out_ref):
    i = pl.program_id(0)
    vt = vt_ref[...]
    whi_blk = whib_ref[...]
    p = pq_ref[1:2, pl.ds(i * BI, BI)]
    q = pq_ref[0:1, :]
    e = p.reshape(BI, 1) + q

    av = (1.0 - ALPHA) * a_ref[...]
    KG = 8
    for k0 in range(0, OUT_F, KG):
        t = None
        for k in range(k0, k0 + KG):
            z = whi_blk[:, k:k + 1] + vt[k:k + 1, :]
            r = av[0, k] * jnp.maximum(z, 0.0)
            t = r if t is None else t + r
        e = e + t

    mask = adj_ref[...] != 0.0
    neg_inf = jnp.float32(-jnp.inf)
    m = jnp.max(jnp.where(mask, e, neg_inf), axis=1, keepdims=True)
    ew = jnp.where(mask, jnp.exp(e - m), 0.0)
    rowsum = jnp.sum(ew, axis=1, keepdims=True)

    hp = jnp.dot(ew, whi_ref[...], preferred_element_type=jnp.float32)
    hp = hp / rowsum
    out_ref[...] = jnp.where(hp > 0.0, hp, jnp.exp(hp) - 1.0)


@jax.jit
def kernel(input, adj, W, a):
    n = input.shape[0]
    whi, vt, pq = pl.pallas_call(
        _project,
        out_shape=(
            jax.ShapeDtypeStruct((n, OUT_F), jnp.float32),
            jax.ShapeDtypeStruct((OUT_F, n), jnp.float32),
            jax.ShapeDtypeStruct((2, n), jnp.float32),
        ),
    )(input, W, a)
    grid = n // BI
    return pl.pallas_call(
        _attend,
        grid=(grid,),
        in_specs=[
            pl.BlockSpec((n, OUT_F), lambda i: (0, 0)),
            pl.BlockSpec((BI, OUT_F), lambda i: (i, 0)),
            pl.BlockSpec((OUT_F, n), lambda i: (0, 0)),
            pl.BlockSpec((2, n), lambda i: (0, 0)),
            pl.BlockSpec((BI, n), lambda i: (i, 0)),
            pl.BlockSpec((1, OUT_F), lambda i: (0, 0)),
        ],
        out_specs=pl.BlockSpec((BI, OUT_F), lambda i: (i, 0)),
        out_shape=jax.ShapeDtypeStruct((n, OUT_F), jnp.float32),
        compiler_params=pltpu.CompilerParams(
            dimension_semantics=("parallel",)),
    )(whi, whi, vt, pq, adj, a)


# speedup vs baseline: 1.1156x; 1.1156x over previous
import jax
import jax.numpy as jnp
from jax.experimental import pallas as pl
from jax.experimental.pallas import tpu as pltpu

IN_F = 128
OUT_F = 64
ALPHA = 0.2
BI = 256


def _gat_block(x_ref, adj_ref, w_ref, a_ref, out_ref):
    i = pl.program_id(0)
    x = x_ref[...]
    w1 = w_ref[:IN_F, :]
    w2 = w_ref[IN_F:, :]
    a = a_ref[...]

    whi = jnp.dot(x, w1, preferred_element_type=jnp.float32)
    vt = jax.lax.dot_general(w2, x, (((0,), (1,)), ((), ())),
                             preferred_element_type=jnp.float32)
    xi = x_ref[pl.ds(i * BI, BI), :]
    whi_blk = jnp.dot(xi, w1, preferred_element_type=jnp.float32)

    p = jnp.dot(whi_blk, a.T, preferred_element_type=jnp.float32)
    q = jnp.dot(a, vt, preferred_element_type=jnp.float32)

    e = ALPHA * (p + q)

    av = (1.0 - ALPHA) * a
    KG = 8
    for k0 in range(0, OUT_F, KG):
        rs = []
        for k in range(k0, k0 + KG):
            z = whi_blk[:, k:k + 1] + vt[k:k + 1, :]
            rs.append(av[0, k] * jnp.maximum(z, 0.0))
        while len(rs) > 1:
            rs = [rs[m] + rs[m + 1] for m in range(0, len(rs), 2)]
        e = e + rs[0]

    adjb = adj_ref[...]
    mask = adjb != 0.0
    neg_inf = jnp.float32(-jnp.inf)
    m = jnp.max(jnp.where(mask, e, neg_inf), axis=1, keepdims=True)
    ew = adjb * jnp.exp(jnp.minimum(e - m, 0.0))
    rowsum = jnp.sum(ew, axis=1, keepdims=True)

    hp = jnp.dot(ew, whi, preferred_element_type=jnp.float32)
    hp = hp / rowsum
    out_ref[...] = jnp.where(hp > 0.0, hp, jnp.exp(hp) - 1.0)


@jax.jit
def kernel(input, adj, W, a):
    n = input.shape[0]
    grid = n // BI
    return pl.pallas_call(
        _gat_block,
        grid=(grid,),
        in_specs=[
            pl.BlockSpec((n, IN_F), lambda i: (0, 0)),
            pl.BlockSpec((BI, n), lambda i: (i, 0)),
            pl.BlockSpec((2 * IN_F, OUT_F), lambda i: (0, 0)),
            pl.BlockSpec((1, OUT_F), lambda i: (0, 0)),
        ],
        out_specs=pl.BlockSpec((BI, OUT_F), lambda i: (i, 0)),
        out_shape=jax.ShapeDtypeStruct((n, OUT_F), jnp.float32),
        compiler_params=pltpu.CompilerParams(
            dimension_semantics=("parallel",),
            vmem_limit_bytes=120 * 1024 * 1024),
    )(input, adj, W, a)


# speedup vs baseline: 1.1398x; 1.0217x over previous
import jax
import jax.numpy as jnp
from jax.experimental import pallas as pl
from jax.experimental.pallas import tpu as pltpu

IN_F = 128
OUT_F = 64
ALPHA = 0.2
BI = 256


def _gat_block(x_ref, adj_ref, w_ref, a_ref, out_ref):
    i = pl.program_id(0)
    x = x_ref[...]
    w1 = w_ref[:IN_F, :]
    w2 = w_ref[IN_F:, :]
    a = a_ref[...]

    whi = jnp.dot(x, w1, preferred_element_type=jnp.float32)
    vt = jax.lax.dot_general(w2, x, (((0,), (1,)), ((), ())),
                             preferred_element_type=jnp.float32)
    xi = x_ref[pl.ds(i * BI, BI), :]
    whi_blk = jnp.dot(xi, w1, preferred_element_type=jnp.float32)

    p = jnp.dot(whi_blk, a.T, preferred_element_type=jnp.float32)
    q = jnp.dot(a, vt, preferred_element_type=jnp.float32)

    e = ALPHA * (p + q)

    av = (1.0 - ALPHA) * a
    KG = 8
    accs = [e, None, None, None]
    for g, k0 in enumerate(range(0, OUT_F, KG)):
        rs = []
        for k in range(k0, k0 + KG):
            z = whi_blk[:, k:k + 1] + vt[k:k + 1, :]
            rs.append(av[0, k] * jnp.maximum(z, 0.0))
        while len(rs) > 1:
            rs = [rs[m] + rs[m + 1] for m in range(0, len(rs), 2)]
        s = g % 2
        accs[s] = rs[0] if accs[s] is None else accs[s] + rs[0]
    e = accs[0] + accs[1]

    adjb = adj_ref[...]
    mask = adjb != 0.0
    neg_inf = jnp.float32(-jnp.inf)
    m = jnp.max(jnp.where(mask, e, neg_inf), axis=1, keepdims=True)
    ew = adjb * jnp.exp(jnp.minimum(e - m, 0.0))
    rowsum = jnp.sum(ew, axis=1, keepdims=True)

    hp = jnp.dot(ew, whi, preferred_element_type=jnp.float32)
    hp = hp / rowsum
    out_ref[...] = jnp.where(hp > 0.0, hp, jnp.exp(hp) - 1.0)


@jax.jit
def kernel(input, adj, W, a):
    n = input.shape[0]
    grid = n // BI
    return pl.pallas_call(
        _gat_block,
        grid=(grid,),
        in_specs=[
            pl.BlockSpec((n, IN_F), lambda i: (0, 0)),
            pl.BlockSpec((BI, n), lambda i: (i, 0)),
            pl.BlockSpec((2 * IN_F, OUT_F), lambda i: (0, 0)),
            pl.BlockSpec((1, OUT_F), lambda i: (0, 0)),
        ],
        out_specs=pl.BlockSpec((BI, OUT_F), lambda i: (i, 0)),
        out_shape=jax.ShapeDtypeStruct((n, OUT_F), jnp.float32),
        compiler_params=pltpu.CompilerParams(
            dimension_semantics=("parallel",),
            vmem_limit_bytes=120 * 1024 * 1024),
    )(input, adj, W, a)
